# Initial kernel scaffold; baseline (speedup 1.0000x reference)
import jax
import jax.numpy as jnp
from jax.experimental import pallas as pl


def kernel(x, edge_index, node_cnt, W1_l, b1, W1_r, a1, W2_l, b2, W2_r, a2):
    raise NotImplementedError("write your pallas kernel here")


# speedup vs baseline: 3.2082x; 3.2082x over previous
import functools

import jax
import jax.numpy as jnp
from jax import lax
from jax.experimental import pallas as pl
from jax.experimental.pallas import tpu as pltpu
from jax.experimental.pallas import tpu_sc as plsc

N = 10000
NPAD = 10240
E = 160000
D = 256
DH = 128
NC = 2
NS = 16
CH = 128
EPAD = NS * 80 * CH
ECHUNKS = EPAD // (NS * CH)
ROWS_PER_TILE = NPAD // NS
ZCOPIES = ROWS_PER_TILE // CH

_mesh = plsc.VectorSubcoreMesh(core_axis_name="c", subcore_axis_name="s",
                               num_cores=NC, num_subcores=NS)


@functools.partial(
    pl.kernel,
    out_type=(
        jax.ShapeDtypeStruct((NPAD, DH), jnp.float32),
        jax.ShapeDtypeStruct((NPAD, DH), jnp.float32),
    ),
    mesh=_mesh,
    scratch_types=[
        pltpu.VMEM((ECHUNKS, CH), jnp.int32),
        pltpu.VMEM((ECHUNKS, CH), jnp.int32),
        pltpu.VMEM((ECHUNKS, CH), jnp.int32),
        pltpu.VMEM((CH, DH), jnp.float32),
        pltpu.VMEM_SHARED((NPAD, DH), jnp.float32),
        pltpu.SemaphoreType.DMA,
    ],
)
def _sc_segment_sum(x2, src_r, dst_r, out0, out1,
                    srcc, dstc, idxc, gbuf, accs, sem):
    c = lax.axis_index("c")
    t = lax.axis_index("s")

    pltpu.sync_copy(src_r.at[pl.ds(t * ECHUNKS, ECHUNKS)], srcc)
    pltpu.sync_copy(dst_r.at[pl.ds(t * ECHUNKS, ECHUNKS)], dstc)

    def _idx_body(k, carry):
        for j in range(CH // 16):
            s = srcc[k, pl.ds(j * 16, 16)]
            idxc[k, pl.ds(j * 16, 16)] = s * 2 + c
        return carry
    lax.fori_loop(0, ECHUNKS, _idx_body, 0)

    def _zero_body(k, carry):
        for j in range(DH // 16):
            gbuf[k, pl.ds(j * 16, 16)] = jnp.zeros((16,), jnp.float32)
        return carry
    lax.fori_loop(0, CH, _zero_body, 0)
    for i in range(ZCOPIES):
        rows_i = pl.ds(t * ROWS_PER_TILE + i * CH, CH)
        pltpu.sync_copy(gbuf, accs.at[rows_i])
    plsc.subcore_barrier()

    def _edge_body(k, carry):
        pltpu.async_copy(x2.at[idxc.at[k]], gbuf, sem).wait()
        pltpu.sync_copy(gbuf, accs.at[dstc.at[k]], add=True)
        return carry
    lax.fori_loop(0, ECHUNKS, _edge_body, 0)
    plsc.subcore_barrier()

    rows = pl.ds(t * ROWS_PER_TILE, ROWS_PER_TILE)

    @pl.when(c == 0)
    def _():
        pltpu.sync_copy(accs.at[rows], out0.at[rows])

    @pl.when(c == 1)
    def _():
        pltpu.sync_copy(accs.at[rows], out1.at[rows])


EB = 4096


def _tc_degree_body(db, ob):
    i = pl.program_id(0)
    d = db[...]
    h = d >> 7
    l = d & 127
    ioh = lax.broadcasted_iota(jnp.int32, (EB, 128), 1)
    oh = (h == ioh).astype(jnp.bfloat16)
    ol = (l == ioh).astype(jnp.bfloat16)
    prod = lax.dot_general(oh, ol, (((0,), (0,)), ((), ())),
                           preferred_element_type=jnp.float32)

    @pl.when(i == 0)
    def _():
        ob[...] = prod

    @pl.when(i != 0)
    def _():
        ob[...] += prod


_tc_degree = pl.pallas_call(
    _tc_degree_body,
    grid=(EPAD // EB,),
    in_specs=[pl.BlockSpec((EB, 1), lambda i: (i, 0))],
    out_specs=pl.BlockSpec((128, 128), lambda i: (0, 0)),
    out_shape=jax.ShapeDtypeStruct((128, 128), jnp.float32),
)


RB = 512


def _tc_layer_body(a0, a1, degb, xb, wl, wr, bb, ab, ob):
    mean = jnp.concatenate([a0[...], a1[...]], axis=1) / degb[...]
    dn = (((1,), (1,)), ((), ()))
    out = (lax.dot_general(mean, wl[...], dn,
                           precision=lax.Precision.HIGHEST,
                           preferred_element_type=jnp.float32)
           + bb[...]
           + lax.dot_general(xb[...], wr[...], dn,
                             precision=lax.Precision.HIGHEST,
                             preferred_element_type=jnp.float32))
    norm = jnp.sqrt(jnp.sum(out * out, axis=-1, keepdims=True))
    out = out / jnp.maximum(norm, 1e-12)
    ob[...] = jnp.where(out >= 0.0, out, ab[...] * out)


_tc_layer = pl.pallas_call(
    _tc_layer_body,
    grid=(NPAD // RB,),
    in_specs=[
        pl.BlockSpec((RB, DH), lambda i: (i, 0)),
        pl.BlockSpec((RB, DH), lambda i: (i, 0)),
        pl.BlockSpec((RB, 1), lambda i: (i, 0)),
        pl.BlockSpec((RB, D), lambda i: (i, 0)),
        pl.BlockSpec((D, D), lambda i: (0, 0)),
        pl.BlockSpec((D, D), lambda i: (0, 0)),
        pl.BlockSpec((1, D), lambda i: (0, 0)),
        pl.BlockSpec((1, D), lambda i: (0, 0)),
    ],
    out_specs=pl.BlockSpec((RB, D), lambda i: (i, 0)),
    out_shape=jax.ShapeDtypeStruct((NPAD, D), jnp.float32),
)


def kernel(x, edge_index, node_cnt, W1_l, b1, W1_r, a1, W2_l, b2, W2_r, a2):
    del node_cnt
    xp = jnp.pad(x, ((0, NPAD - N), (0, 0)))
    src = edge_index[0]
    dst = edge_index[1]
    pad_e = EPAD - E
    srcp = jnp.concatenate([src, jnp.zeros((pad_e,), jnp.int32)])
    dstp = jnp.concatenate([dst, jnp.full((pad_e,), NPAD - 1, jnp.int32)])
    src_r = srcp.reshape(EPAD // CH, CH)
    dst_r = dstp.reshape(EPAD // CH, CH)

    deg2 = _tc_degree(dstp.reshape(EPAD, 1))
    deg_col = jnp.maximum(deg2[:NPAD // 128].reshape(NPAD), 1.0)[:, None]

    agg0, agg1 = _sc_segment_sum(xp.reshape(NPAD * 2, DH), src_r, dst_r)
    h1 = _tc_layer(agg0, agg1, deg_col, xp, W1_l, W1_r,
                   b1.reshape(1, D), a1.reshape(1, D))
    agg0b, agg1b = _sc_segment_sum(h1.reshape(NPAD * 2, DH), src_r, dst_r)
    h2 = _tc_layer(agg0b, agg1b, deg_col, h1, W2_l, W2_r,
                   b2.reshape(1, D), a2.reshape(1, D))
    return h2[:N]


# speedup vs baseline: 3.7234x; 1.1606x over previous
import functools

import jax
import jax.numpy as jnp
from jax import lax
from jax.experimental import pallas as pl
from jax.experimental.pallas import tpu as pltpu
from jax.experimental.pallas import tpu_sc as plsc

N = 10000
NPAD = 10240
E = 160000
D = 256
DH = 128
NC = 2
NS = 16
CH = 128
ECHUNKS = 80
EPAD = NS * ECHUNKS * CH
ROWS_PER_TILE = NPAD // NS
ZCOPIES = ROWS_PER_TILE // CH
NB = 2
W = 16
NWIN = ECHUNKS // W
NSLOT = 3

_mesh = plsc.VectorSubcoreMesh(core_axis_name="c", subcore_axis_name="s",
                               num_cores=NC, num_subcores=NS)


@functools.partial(
    pl.kernel,
    out_type=(
        jax.ShapeDtypeStruct((NPAD, DH), jnp.float32),
        jax.ShapeDtypeStruct((NPAD, DH), jnp.float32),
    ),
    mesh=_mesh,
    scratch_types=[
        pltpu.VMEM((NSLOT * W, CH), jnp.int32),
        pltpu.VMEM((NSLOT * W, CH), jnp.int32),
        pltpu.VMEM((NB * CH, DH), jnp.float32),
        pltpu.VMEM_SHARED((NPAD, DH), jnp.float32),
        pltpu.SemaphoreType.DMA((NB,)),
        pltpu.SemaphoreType.DMA((NSLOT,)),
        pltpu.SemaphoreType.DMA((NSLOT,)),
    ],
)
def _sc_segment_sum(x2, src_r, dst_r, out0, out1,
                    dstc, idxc, gbuf, accs, gsem, ssem, dsem):
    c = lax.axis_index("c")
    t = lax.axis_index("s")

    def _win_rows(w):
        return pl.ds(t * ECHUNKS + w * W, W)

    def _slot(w):
        return lax.rem(w, NSLOT)

    def _issue_win(w):
        sl = _slot(w)
        rows = pl.ds(sl * W, W)
        pltpu.async_copy(src_r.at[_win_rows(w)], idxc.at[rows], ssem.at[sl])
        pltpu.async_copy(dst_r.at[_win_rows(w)], dstc.at[rows], dsem.at[sl])

    def _prep_win(w):
        sl = _slot(w)
        rows = pl.ds(sl * W, W)
        pltpu.make_async_copy(src_r.at[_win_rows(w)], idxc.at[rows],
                              ssem.at[sl]).wait()
        pltpu.make_async_copy(dst_r.at[_win_rows(w)], dstc.at[rows],
                              dsem.at[sl]).wait()

        def _idx_body(r, carry):
            for j in range(CH // 16):
                s = idxc[sl * W + r, pl.ds(j * 16, 16)]
                idxc[sl * W + r, pl.ds(j * 16, 16)] = s * 2 + c
            return carry
        lax.fori_loop(0, W, _idx_body, 0)

    def _gb(b):
        return gbuf.at[pl.ds(b * CH, CH)]

    def _idx_row(w, j):
        return idxc.at[_slot(w) * W + j]

    def _zero_body(k, carry):
        for j in range(DH // 16):
            gbuf[k, pl.ds(j * 16, 16)] = jnp.zeros((16,), jnp.float32)
        return carry
    lax.fori_loop(0, CH, _zero_body, 0)
    for i in range(ZCOPIES):
        rows_i = pl.ds(t * ROWS_PER_TILE + i * CH, CH)
        pltpu.sync_copy(gbuf.at[pl.ds(0, CH)], accs.at[rows_i])
    plsc.subcore_barrier()

    _issue_win(0)
    _prep_win(0)
    _issue_win(1)
    for b in range(NB):
        pltpu.async_copy(x2.at[_idx_row(0, b)], _gb(b), gsem.at[b])

    def _win_body(w, carry):
        @pl.when(w + 1 < NWIN)
        def _():
            _prep_win(w + 1)

        @pl.when(w + 2 < NWIN)
        def _():
            _issue_win(w + 2)

        for j in range(W):
            k = w * W + j
            b = j % NB
            pltpu.make_async_copy(x2.at[_idx_row(w, j)], _gb(b),
                                  gsem.at[b]).wait()
            pltpu.sync_copy(_gb(b), accs.at[dstc.at[_slot(w) * W + j]],
                            add=True)

            @pl.when(k + NB < ECHUNKS)
            def _():
                wn = w + 1 if j + NB >= W else w
                jn = (j + NB) % W
                pltpu.async_copy(x2.at[_idx_row(wn, jn)], _gb(b), gsem.at[b])
        return carry
    lax.fori_loop(0, NWIN, _win_body, 0)
    plsc.subcore_barrier()

    rows = pl.ds(t * ROWS_PER_TILE, ROWS_PER_TILE)

    @pl.when(c == 0)
    def _():
        pltpu.sync_copy(accs.at[rows], out0.at[rows])

    @pl.when(c == 1)
    def _():
        pltpu.sync_copy(accs.at[rows], out1.at[rows])


EB = 4096


def _tc_degree_body(db, ob):
    i = pl.program_id(0)
    d = db[...]
    h = d >> 7
    l = d & 127
    ioh = lax.broadcasted_iota(jnp.int32, (EB, 128), 1)
    oh = (h == ioh).astype(jnp.bfloat16)
    ol = (l == ioh).astype(jnp.bfloat16)
    prod = lax.dot_general(oh, ol, (((0,), (0,)), ((), ())),
                           preferred_element_type=jnp.float32)

    @pl.when(i == 0)
    def _():
        ob[...] = prod

    @pl.when(i != 0)
    def _():
        ob[...] += prod


_tc_degree = pl.pallas_call(
    _tc_degree_body,
    grid=(EPAD // EB,),
    in_specs=[pl.BlockSpec((EB, 1), lambda i: (i, 0))],
    out_specs=pl.BlockSpec((128, 128), lambda i: (0, 0)),
    out_shape=jax.ShapeDtypeStruct((128, 128), jnp.float32),
)


RB = 512


def _tc_layer_body(a0, a1, degb, xb, wl, wr, bb, ab, ob):
    mean = jnp.concatenate([a0[...], a1[...]], axis=1) / degb[...]
    dn = (((1,), (1,)), ((), ()))
    out = (lax.dot_general(mean, wl[...], dn,
                           precision=lax.Precision.HIGHEST,
                           preferred_element_type=jnp.float32)
           + bb[...]
           + lax.dot_general(xb[...], wr[...], dn,
                             precision=lax.Precision.HIGHEST,
                             preferred_element_type=jnp.float32))
    norm = jnp.sqrt(jnp.sum(out * out, axis=-1, keepdims=True))
    out = out / jnp.maximum(norm, 1e-12)
    ob[...] = jnp.where(out >= 0.0, out, ab[...] * out)


_tc_layer = pl.pallas_call(
    _tc_layer_body,
    grid=(NPAD // RB,),
    in_specs=[
        pl.BlockSpec((RB, DH), lambda i: (i, 0)),
        pl.BlockSpec((RB, DH), lambda i: (i, 0)),
        pl.BlockSpec((RB, 1), lambda i: (i, 0)),
        pl.BlockSpec((RB, D), lambda i: (i, 0)),
        pl.BlockSpec((D, D), lambda i: (0, 0)),
        pl.BlockSpec((D, D), lambda i: (0, 0)),
        pl.BlockSpec((1, D), lambda i: (0, 0)),
        pl.BlockSpec((1, D), lambda i: (0, 0)),
    ],
    out_specs=pl.BlockSpec((RB, D), lambda i: (i, 0)),
    out_shape=jax.ShapeDtypeStruct((NPAD, D), jnp.float32),
)


def kernel(x, edge_index, node_cnt, W1_l, b1, W1_r, a1, W2_l, b2, W2_r, a2):
    del node_cnt
    xp = jnp.pad(x, ((0, NPAD - N), (0, 0)))
    src = edge_index[0]
    dst = edge_index[1]
    pad_e = EPAD - E
    srcp = jnp.concatenate([src, jnp.zeros((pad_e,), jnp.int32)])
    dstp = jnp.concatenate([dst, jnp.full((pad_e,), NPAD - 1, jnp.int32)])
    src_r = srcp.reshape(EPAD // CH, CH)
    dst_r = dstp.reshape(EPAD // CH, CH)

    deg2 = _tc_degree(dstp.reshape(EPAD, 1))
    deg_col = jnp.maximum(deg2[:NPAD // 128].reshape(NPAD), 1.0)[:, None]

    agg0, agg1 = _sc_segment_sum(xp.reshape(NPAD * 2, DH), src_r, dst_r)
    h1 = _tc_layer(agg0, agg1, deg_col, xp, W1_l, W1_r,
                   b1.reshape(1, D), a1.reshape(1, D))
    agg0b, agg1b = _sc_segment_sum(h1.reshape(NPAD * 2, DH), src_r, dst_r)
    h2 = _tc_layer(agg0b, agg1b, deg_col, h1, W2_l, W2_r,
                   b2.reshape(1, D), a2.reshape(1, D))
    return h2[:N]
